# baseline (speedup 1.0000x reference)
import functools

import jax
import jax.numpy as jnp
from jax import lax
from jax.experimental import pallas as pl
from jax.experimental.pallas import tpu as pltpu
from jax.experimental.pallas import tpu_sc as plsc

DIM = 64
B = 16384
LAMBD = 0.1
L = 16


def _sc_make(nc, ns):
    nw = nc * ns
    rpw = B // nw
    ch = 128
    nch = rpw // ch
    ngrp = rpw // L

    mesh = plsc.VectorSubcoreMesh(core_axis_name="c", subcore_axis_name="s")
    vec_f32 = jax.ShapeDtypeStruct((B,), jnp.float32)

    @functools.partial(
        pl.kernel,
        mesh=mesh,
        out_type=(vec_f32, vec_f32, vec_f32, vec_f32),
        compiler_params=pltpu.CompilerParams(
            needs_layout_passes=False, use_tc_tiling_on_sc=False),
        scratch_types=[
            pltpu.VMEM((nch, ch), jnp.int32),
            pltpu.VMEM((nch, ch), jnp.int32),
            pltpu.VMEM((rpw, DIM), jnp.float32),
            pltpu.VMEM((rpw, DIM), jnp.float32),
            pltpu.VMEM((rpw,), jnp.float32),
            pltpu.VMEM((rpw,), jnp.float32),
            pltpu.VMEM((rpw,), jnp.float32),
            pltpu.VMEM((rpw,), jnp.float32),
            pltpu.VMEM((L * L,), jnp.float32),
            pltpu.VMEM((L * L,), jnp.float32),
            pltpu.VMEM((L * L,), jnp.float32),
            pltpu.SemaphoreType.DMA,
        ],
    )
    def sc_fn(src_hbm, ctx_hbm, psrc_hbm, pers_hbm,
              node_hbm, noise_hbm, base_hbm,
              ab_out, aa_out, bb_out, rd_out,
              idx_a, idx_b, a_rows, b_rows, ab_v, aa_v, bb_v, rd_v,
              pab, paa, pbb, sem):
        wid = lax.axis_index("s") * nc + lax.axis_index("c")
        base = pl.multiple_of(wid * rpw, rpw)
        iota = lax.iota(jnp.int32, L)
        col0 = iota * L

        def load_rows(ia_hbm, ib_hbm, ta_hbm, tb_hbm):
            for j in range(nch):
                off = pl.multiple_of(base + j * ch, ch)
                pltpu.sync_copy(ia_hbm.at[pl.ds(off, ch)], idx_a.at[j])
                pltpu.sync_copy(ib_hbm.at[pl.ds(off, ch)], idx_b.at[j])
            cps = []
            for j in range(nch):
                cps.append(pltpu.async_copy(
                    ta_hbm.at[idx_a.at[j]], a_rows.at[pl.ds(j * ch, ch)], sem))
                cps.append(pltpu.async_copy(
                    tb_hbm.at[idx_b.at[j]], b_rows.at[pl.ds(j * ch, ch)], sem))
            for c in cps:
                c.wait()

        load_rows(src_hbm, ctx_hbm, node_hbm, noise_hbm)

        zero = jnp.zeros((L,), jnp.float32)

        def main_group(g, _):
            rbase = g * L
            for r in range(L):
                row = rbase + r
                ab_p, aa_p, bb_p = zero, zero, zero
                for q in range(DIM // L):
                    a = a_rows[row, pl.ds(q * L, L)]
                    b = b_rows[row, pl.ds(q * L, L)]
                    ab_p += a * b
                    aa_p += a * a
                    bb_p += b * b
                pab[pl.ds(r * L, L)] = ab_p
                paa[pl.ds(r * L, L)] = aa_p
                pbb[pl.ds(r * L, L)] = bb_p
            ab, aa, bb = zero, zero, zero
            for c in range(L):
                cidx = col0 + c
                ab += plsc.load_gather(pab, [cidx])
                aa += plsc.load_gather(paa, [cidx])
                bb += plsc.load_gather(pbb, [cidx])
            off = pl.multiple_of(rbase, L)
            ab_v[pl.ds(off, L)] = ab
            aa_v[pl.ds(off, L)] = aa
            bb_v[pl.ds(off, L)] = bb
            return 0

        lax.fori_loop(0, ngrp, main_group, 0)

        load_rows(psrc_hbm, pers_hbm, node_hbm, base_hbm)

        def reg_group(g, _):
            rbase = g * L
            for r in range(L):
                row = rbase + r
                rd_p = zero
                for q in range(DIM // L):
                    a = a_rows[row, pl.ds(q * L, L)]
                    b = b_rows[row, pl.ds(q * L, L)]
                    rd_p += a * b
                pab[pl.ds(r * L, L)] = rd_p
            rd = zero
            for c in range(L):
                rd += plsc.load_gather(pab, [col0 + c])
            rd_v[pl.ds(pl.multiple_of(rbase, L), L)] = rd
            return 0

        lax.fori_loop(0, ngrp, reg_group, 0)

        obase = pl.multiple_of(base, 8)
        pltpu.sync_copy(ab_v, ab_out.at[pl.ds(obase, rpw)])
        pltpu.sync_copy(aa_v, aa_out.at[pl.ds(obase, rpw)])
        pltpu.sync_copy(bb_v, bb_out.at[pl.ds(obase, rpw)])
        pltpu.sync_copy(rd_v, rd_out.at[pl.ds(obase, rpw)])

    return sc_fn


def _tc_loss(t_ref, ab_ref, aa_ref, bb_ref, rd_ref, out_ref):
    ab = ab_ref[...]
    na = jnp.maximum(jnp.sqrt(aa_ref[...]), 1e-12)
    nb = jnp.maximum(jnp.sqrt(bb_ref[...]), 1e-12)
    s = jax.nn.sigmoid(ab / (na * nb))
    t = t_ref[...]
    main = t * jnp.log(s) + (1.0 - t) * jnp.log(1.0 - s)
    r = jnp.clip(rd_ref[...], -15.0, 15.0)
    rl = jnp.log(jax.nn.sigmoid(r))
    loss = -(jnp.sum(main) / B) - LAMBD * (jnp.sum(rl) / B)
    out_ref[...] = jnp.full((1, 1), loss, jnp.float32)


def kernel(sources, contexts, targets, personas, pure_sources,
           node_embedding, node_noise_embedding, base_node_embedding):
    info = plsc.get_sparse_core_info()
    sc_fn = _sc_make(info.num_cores, info.num_subcores)
    ab, aa, bb, rd = sc_fn(
        sources.astype(jnp.int32), contexts.astype(jnp.int32),
        pure_sources.astype(jnp.int32), personas.astype(jnp.int32),
        node_embedding, node_noise_embedding, base_node_embedding)
    sh = (B // 128, 128)
    loss = pl.pallas_call(
        _tc_loss,
        out_shape=jax.ShapeDtypeStruct((1, 1), jnp.float32),
    )(targets.reshape(sh), ab.reshape(sh), aa.reshape(sh),
      bb.reshape(sh), rd.reshape(sh))
    return loss[0, 0]


# speedup vs baseline: 1.9471x; 1.9471x over previous
import functools

import jax
import jax.numpy as jnp
from jax import lax
from jax.experimental import pallas as pl
from jax.experimental.pallas import tpu as pltpu
from jax.experimental.pallas import tpu_sc as plsc

DIM = 64
B = 16384
LAMBD = 0.1
L = 16
CK = 32


def _sc_make(nc, ns):
    nw = nc * ns
    rpw = B // nw
    nchunk = rpw // CK

    mesh = plsc.VectorSubcoreMesh(core_axis_name="c", subcore_axis_name="s")
    vec_f32 = jax.ShapeDtypeStruct((B,), jnp.float32)

    @functools.partial(
        pl.kernel,
        mesh=mesh,
        out_type=(vec_f32, vec_f32, vec_f32, vec_f32),
        compiler_params=pltpu.CompilerParams(
            needs_layout_passes=False, use_tc_tiling_on_sc=True),
        scratch_types=[
            pltpu.VMEM((rpw,), jnp.int32),
            pltpu.VMEM((rpw,), jnp.int32),
            pltpu.VMEM((CK, 8, DIM), jnp.float32),
            pltpu.VMEM((CK, 8, DIM), jnp.float32),
            pltpu.VMEM((rpw,), jnp.float32),
            pltpu.VMEM((rpw,), jnp.float32),
            pltpu.VMEM((rpw,), jnp.float32),
            pltpu.VMEM((rpw,), jnp.float32),
            pltpu.VMEM((L * L,), jnp.float32),
            pltpu.VMEM((L * L,), jnp.float32),
            pltpu.VMEM((L * L,), jnp.float32),
            pltpu.SemaphoreType.DMA,
        ],
    )
    def sc_fn(src_hbm, ctx_hbm, psrc_hbm, pers_hbm,
              node_hbm, noise_hbm, base_hbm,
              ab_out, aa_out, bb_out, rd_out,
              idx_a, idx_b, a_buf, b_buf, ab_v, aa_v, bb_v, rd_v,
              pab, paa, pbb, sem):
        wid = lax.axis_index("s") * nc + lax.axis_index("c")
        base = pl.multiple_of(wid * rpw, rpw)
        col0 = lax.iota(jnp.int32, L) * L
        zero = jnp.zeros((L,), jnp.float32)

        def load_indices(ia_hbm, ib_hbm):
            pltpu.sync_copy(ia_hbm.at[pl.ds(base, rpw)], idx_a)
            pltpu.sync_copy(ib_hbm.at[pl.ds(base, rpw)], idx_b)

        def phase(ta_hbm, tb_hbm, with_norms, out_main, out_aa, out_bb):
            def chunk_body(t, _):
                coff = pl.multiple_of(t * CK, CK)
                cps = []
                subs = []
                for half in range(CK // L):
                    hoff = pl.multiple_of(coff + half * L, L)
                    iav = idx_a[pl.ds(hoff, L)]
                    ibv = idx_b[pl.ds(hoff, L)]
                    for r in range(L):
                        k = half * L + r
                        ia = iav[r]
                        ib = ibv[r]
                        subs.append((ia & 7, ib & 7))
                        cps.append(pltpu.async_copy(
                            ta_hbm.at[pl.ds(ia >> 3, 1)],
                            a_buf.at[pl.ds(k, 1)], sem))
                        cps.append(pltpu.async_copy(
                            tb_hbm.at[pl.ds(ib >> 3, 1)],
                            b_buf.at[pl.ds(k, 1)], sem))
                for c in cps:
                    c.wait()
                for half in range(CK // L):
                    for r in range(L):
                        k = half * L + r
                        sa, sb = subs[k]
                        ab_p, aa_p, bb_p = zero, zero, zero
                        for q in range(DIM // L):
                            a = a_buf[k, sa, pl.ds(q * L, L)]
                            b = b_buf[k, sb, pl.ds(q * L, L)]
                            ab_p += a * b
                            if with_norms:
                                aa_p += a * a
                                bb_p += b * b
                        pab[pl.ds(r * L, L)] = ab_p
                        if with_norms:
                            paa[pl.ds(r * L, L)] = aa_p
                            pbb[pl.ds(r * L, L)] = bb_p
                    ab, aa, bb = zero, zero, zero
                    for c in range(L):
                        cidx = col0 + c
                        ab += plsc.load_gather(pab, [cidx])
                        if with_norms:
                            aa += plsc.load_gather(paa, [cidx])
                            bb += plsc.load_gather(pbb, [cidx])
                    off = pl.multiple_of(coff + half * L, L)
                    out_main[pl.ds(off, L)] = ab
                    if with_norms:
                        out_aa[pl.ds(off, L)] = aa
                        out_bb[pl.ds(off, L)] = bb
                return 0

            lax.fori_loop(0, nchunk, chunk_body, 0)

        load_indices(src_hbm, ctx_hbm)
        phase(node_hbm, noise_hbm, True, ab_v, aa_v, bb_v)

        load_indices(psrc_hbm, pers_hbm)
        phase(node_hbm, base_hbm, False, rd_v, None, None)

        obase = pl.multiple_of(base, 8)
        pltpu.sync_copy(ab_v, ab_out.at[pl.ds(obase, rpw)])
        pltpu.sync_copy(aa_v, aa_out.at[pl.ds(obase, rpw)])
        pltpu.sync_copy(bb_v, bb_out.at[pl.ds(obase, rpw)])
        pltpu.sync_copy(rd_v, rd_out.at[pl.ds(obase, rpw)])

    return sc_fn


def _tc_loss(t_ref, ab_ref, aa_ref, bb_ref, rd_ref, out_ref):
    ab = ab_ref[...]
    na = jnp.maximum(jnp.sqrt(aa_ref[...]), 1e-12)
    nb = jnp.maximum(jnp.sqrt(bb_ref[...]), 1e-12)
    s = jax.nn.sigmoid(ab / (na * nb))
    t = t_ref[...]
    main = t * jnp.log(s) + (1.0 - t) * jnp.log(1.0 - s)
    r = jnp.clip(rd_ref[...], -15.0, 15.0)
    rl = jnp.log(jax.nn.sigmoid(r))
    loss = -(jnp.sum(main) / B) - LAMBD * (jnp.sum(rl) / B)
    out_ref[...] = jnp.full((1, 1), loss, jnp.float32)


def kernel(sources, contexts, targets, personas, pure_sources,
           node_embedding, node_noise_embedding, base_node_embedding):
    info = plsc.get_sparse_core_info()
    sc_fn = _sc_make(info.num_cores, info.num_subcores)
    node3 = node_embedding.reshape(-1, 8, DIM)
    noise3 = node_noise_embedding.reshape(-1, 8, DIM)
    base3 = base_node_embedding.reshape(-1, 8, DIM)
    ab, aa, bb, rd = sc_fn(
        sources.astype(jnp.int32), contexts.astype(jnp.int32),
        pure_sources.astype(jnp.int32), personas.astype(jnp.int32),
        node3, noise3, base3)
    sh = (B // 128, 128)
    loss = pl.pallas_call(
        _tc_loss,
        out_shape=jax.ShapeDtypeStruct((1, 1), jnp.float32),
    )(targets.reshape(sh), ab.reshape(sh), aa.reshape(sh),
      bb.reshape(sh), rd.reshape(sh))
    return loss[0, 0]


# speedup vs baseline: 1.9864x; 1.0202x over previous
import functools

import jax
import jax.numpy as jnp
from jax import lax
from jax.experimental import pallas as pl
from jax.experimental.pallas import tpu as pltpu
from jax.experimental.pallas import tpu_sc as plsc

DIM = 64
B = 16384
LAMBD = 0.1
L = 16
CK = 16


def _sc_make(nc, ns):
    nw = nc * ns
    rpw = B // nw
    nchunk = rpw // CK

    mesh = plsc.VectorSubcoreMesh(core_axis_name="c", subcore_axis_name="s")
    vec_f32 = jax.ShapeDtypeStruct((B,), jnp.float32)

    @functools.partial(
        pl.kernel,
        mesh=mesh,
        out_type=(vec_f32, vec_f32, vec_f32, vec_f32),
        compiler_params=pltpu.CompilerParams(
            needs_layout_passes=False, use_tc_tiling_on_sc=True),
        scratch_types=[
            pltpu.VMEM((rpw,), jnp.int32),
            pltpu.VMEM((rpw,), jnp.int32),
            pltpu.VMEM((2 * CK, 8, DIM), jnp.float32),
            pltpu.VMEM((2 * CK, 8, DIM), jnp.float32),
            pltpu.VMEM((rpw,), jnp.float32),
            pltpu.VMEM((rpw,), jnp.float32),
            pltpu.VMEM((rpw,), jnp.float32),
            pltpu.VMEM((rpw,), jnp.float32),
            pltpu.VMEM((L * L,), jnp.float32),
            pltpu.VMEM((L * L,), jnp.float32),
            pltpu.VMEM((L * L,), jnp.float32),
            pltpu.SemaphoreType.DMA,
            pltpu.SemaphoreType.DMA,
        ],
    )
    def sc_fn(src_hbm, ctx_hbm, psrc_hbm, pers_hbm,
              node_hbm, noise_hbm, base_hbm,
              ab_out, aa_out, bb_out, rd_out,
              idx_a, idx_b, a_buf, b_buf, ab_v, aa_v, bb_v, rd_v,
              pab, paa, pbb, sem0, sem1):
        wid = lax.axis_index("s") * nc + lax.axis_index("c")
        base = pl.multiple_of(wid * rpw, rpw)
        col0 = lax.iota(jnp.int32, L) * L
        zero = jnp.zeros((L,), jnp.float32)

        def load_indices(ia_hbm, ib_hbm):
            pltpu.sync_copy(ia_hbm.at[pl.ds(base, rpw)], idx_a)
            pltpu.sync_copy(ib_hbm.at[pl.ds(base, rpw)], idx_b)

        def phase(ta_hbm, tb_hbm, with_norms, out_main, out_aa, out_bb):
            def fire(t, boff, sem):
                coff = t * CK
                iav = idx_a[pl.ds(coff, CK)]
                ibv = idx_b[pl.ds(coff, CK)]
                for k in range(CK):
                    ia = iav[k]
                    ib = ibv[k]
                    pltpu.async_copy(
                        ta_hbm.at[pl.ds(ia >> 3, 1)],
                        a_buf.at[pl.ds(boff + k, 1)], sem)
                    pltpu.async_copy(
                        tb_hbm.at[pl.ds(ib >> 3, 1)],
                        b_buf.at[pl.ds(boff + k, 1)], sem)

            def drain(boff, sem):
                pltpu.make_async_copy(
                    ta_hbm.at[pl.ds(0, CK)],
                    a_buf.at[pl.ds(boff, CK)], sem).wait()
                pltpu.make_async_copy(
                    ta_hbm.at[pl.ds(0, CK)],
                    b_buf.at[pl.ds(boff, CK)], sem).wait()

            def compute(t, boff):
                coff = t * CK
                iav = idx_a[pl.ds(coff, CK)]
                ibv = idx_b[pl.ds(coff, CK)]
                for r in range(L):
                    sa = iav[r] & 7
                    sb = ibv[r] & 7
                    ab_p, aa_p, bb_p = zero, zero, zero
                    for q in range(DIM // L):
                        a = a_buf[boff + r, sa, pl.ds(q * L, L)]
                        b = b_buf[boff + r, sb, pl.ds(q * L, L)]
                        ab_p += a * b
                        if with_norms:
                            aa_p += a * a
                            bb_p += b * b
                    pab[pl.ds(r * L, L)] = ab_p
                    if with_norms:
                        paa[pl.ds(r * L, L)] = aa_p
                        pbb[pl.ds(r * L, L)] = bb_p
                ab, aa, bb = zero, zero, zero
                for c in range(L):
                    cidx = col0 + c
                    ab += plsc.load_gather(pab, [cidx])
                    if with_norms:
                        aa += plsc.load_gather(paa, [cidx])
                        bb += plsc.load_gather(pbb, [cidx])
                out_main[pl.ds(coff, L)] = ab
                if with_norms:
                    out_aa[pl.ds(coff, L)] = aa
                    out_bb[pl.ds(coff, L)] = bb

            fire(0, 0, sem0)

            def pair_body(tp, _):
                t0 = tp * 2
                t1 = t0 + 1
                fire(t1, CK, sem1)
                drain(0, sem0)
                compute(t0, 0)

                @pl.when(t1 + 1 < nchunk)
                def _():
                    fire(t1 + 1, 0, sem0)

                drain(CK, sem1)
                compute(t1, CK)
                return 0

            lax.fori_loop(0, nchunk // 2, pair_body, 0)

        load_indices(src_hbm, ctx_hbm)
        phase(node_hbm, noise_hbm, True, ab_v, aa_v, bb_v)

        load_indices(psrc_hbm, pers_hbm)
        phase(node_hbm, base_hbm, False, rd_v, None, None)

        obase = pl.multiple_of(base, 8)
        pltpu.sync_copy(ab_v, ab_out.at[pl.ds(obase, rpw)])
        pltpu.sync_copy(aa_v, aa_out.at[pl.ds(obase, rpw)])
        pltpu.sync_copy(bb_v, bb_out.at[pl.ds(obase, rpw)])
        pltpu.sync_copy(rd_v, rd_out.at[pl.ds(obase, rpw)])

    return sc_fn


def _tc_loss(t_ref, ab_ref, aa_ref, bb_ref, rd_ref, out_ref):
    ab = ab_ref[...]
    na = jnp.maximum(jnp.sqrt(aa_ref[...]), 1e-12)
    nb = jnp.maximum(jnp.sqrt(bb_ref[...]), 1e-12)
    s = jax.nn.sigmoid(ab / (na * nb))
    t = t_ref[...]
    main = t * jnp.log(s) + (1.0 - t) * jnp.log(1.0 - s)
    r = jnp.clip(rd_ref[...], -15.0, 15.0)
    rl = jnp.log(jax.nn.sigmoid(r))
    loss = -(jnp.sum(main) / B) - LAMBD * (jnp.sum(rl) / B)
    out_ref[...] = jnp.full((1, 1), loss, jnp.float32)


def kernel(sources, contexts, targets, personas, pure_sources,
           node_embedding, node_noise_embedding, base_node_embedding):
    info = plsc.get_sparse_core_info()
    sc_fn = _sc_make(info.num_cores, info.num_subcores)
    node3 = node_embedding.reshape(-1, 8, DIM)
    noise3 = node_noise_embedding.reshape(-1, 8, DIM)
    base3 = base_node_embedding.reshape(-1, 8, DIM)
    ab, aa, bb, rd = sc_fn(
        sources.astype(jnp.int32), contexts.astype(jnp.int32),
        pure_sources.astype(jnp.int32), personas.astype(jnp.int32),
        node3, noise3, base3)
    sh = (B // 128, 128)
    loss = pl.pallas_call(
        _tc_loss,
        out_shape=jax.ShapeDtypeStruct((1, 1), jnp.float32),
    )(targets.reshape(sh), ab.reshape(sh), aa.reshape(sh),
      bb.reshape(sh), rd.reshape(sh))
    return loss[0, 0]
